# baseline (speedup 1.0000x reference)
import jax
import jax.numpy as jnp
from jax.experimental import pallas as pl
from jax.experimental.pallas import tpu as pltpu

_P = 5
_PR = _P // 2

_NCHUNK = 4


def _compute(x):
    h = x.shape[1]
    row = jax.lax.broadcasted_iota(jnp.int32, (1, h, 1), 1)
    acc = jnp.zeros_like(x)
    nvalid = jnp.zeros((1, h, 1), dtype=x.dtype)
    for off in range(-_PR, _PR + 1):
        m = row + off
        ok = jnp.logical_and(m >= _PR, m <= h - 1 - _PR)
        acc = acc + jnp.where(ok, x, 0.0)
        nvalid = nvalid + ok.astype(x.dtype)
    acc = acc * jnp.array(_P, x.dtype)
    counting = nvalid * jnp.array(_P, x.dtype)
    return acc / counting


def _body(x_hbm, out_hbm, in0, in1, out0, out1, in_sems, out_sems):
    n = x_hbm.shape[0]
    chunk = n // _NCHUNK
    in_bufs = (in0, in1)
    out_bufs = (out0, out1)

    def copy_in(i):
        return pltpu.make_async_copy(
            x_hbm.at[pl.ds(i * chunk, chunk)], in_bufs[i % 2], in_sems.at[i % 2])

    def copy_out(i):
        return pltpu.make_async_copy(
            out_bufs[i % 2], out_hbm.at[pl.ds(i * chunk, chunk)], out_sems.at[i % 2])

    copy_in(0).start()
    for i in range(_NCHUNK):
        if i + 1 < _NCHUNK:
            copy_in(i + 1).start()
        copy_in(i).wait()
        if i >= 2:
            copy_out(i - 2).wait()
        out_bufs[i % 2][...] = _compute(in_bufs[i % 2][...])
        copy_out(i).start()
    copy_out(_NCHUNK - 2).wait()
    copy_out(_NCHUNK - 1).wait()


def kernel(x):
    B, C, H, W = x.shape
    n = B * C
    chunk = n // _NCHUNK
    xf = x.reshape(n, H, W)
    out = pl.pallas_call(
        _body,
        in_specs=[pl.BlockSpec(memory_space=pl.ANY)],
        out_specs=pl.BlockSpec(memory_space=pl.ANY),
        out_shape=jax.ShapeDtypeStruct((n, H, W), x.dtype),
        scratch_shapes=[
            pltpu.VMEM((chunk, H, W), x.dtype),
            pltpu.VMEM((chunk, H, W), x.dtype),
            pltpu.VMEM((chunk, H, W), x.dtype),
            pltpu.VMEM((chunk, H, W), x.dtype),
            pltpu.SemaphoreType.DMA((2,)),
            pltpu.SemaphoreType.DMA((2,)),
        ],
    )(xf)
    return out.reshape(B, C, H, W)


# speedup vs baseline: 1.2132x; 1.2132x over previous
import jax
import jax.numpy as jnp
from jax.experimental import pallas as pl
from jax.experimental.pallas import tpu as pltpu

_P = 5
_PR = _P // 2

_NCHUNK = 2


def _compute(x):
    h = x.shape[1]
    row = jax.lax.broadcasted_iota(jnp.int32, (1, h, 1), 1)
    acc = jnp.zeros_like(x)
    nvalid = jnp.zeros((1, h, 1), dtype=x.dtype)
    for off in range(-_PR, _PR + 1):
        m = row + off
        ok = jnp.logical_and(m >= _PR, m <= h - 1 - _PR)
        acc = acc + jnp.where(ok, x, 0.0)
        nvalid = nvalid + ok.astype(x.dtype)
    acc = acc * jnp.array(_P, x.dtype)
    counting = nvalid * jnp.array(_P, x.dtype)
    return acc / counting


def _body(x_hbm, out_hbm, in0, in1, out0, out1, in_sems, out_sems):
    n = x_hbm.shape[0]
    chunk = n // _NCHUNK
    in_bufs = (in0, in1)
    out_bufs = (out0, out1)

    def copy_in(i):
        return pltpu.make_async_copy(
            x_hbm.at[pl.ds(i * chunk, chunk)], in_bufs[i % 2], in_sems.at[i % 2])

    def copy_out(i):
        return pltpu.make_async_copy(
            out_bufs[i % 2], out_hbm.at[pl.ds(i * chunk, chunk)], out_sems.at[i % 2])

    copy_in(0).start()
    for i in range(_NCHUNK):
        if i + 1 < _NCHUNK:
            copy_in(i + 1).start()
        copy_in(i).wait()
        if i >= 2:
            copy_out(i - 2).wait()
        out_bufs[i % 2][...] = _compute(in_bufs[i % 2][...])
        copy_out(i).start()
    copy_out(_NCHUNK - 2).wait()
    copy_out(_NCHUNK - 1).wait()


def kernel(x):
    B, C, H, W = x.shape
    n = B * C
    chunk = n // _NCHUNK
    xf = x.reshape(n, H, W)
    out = pl.pallas_call(
        _body,
        in_specs=[pl.BlockSpec(memory_space=pl.ANY)],
        out_specs=pl.BlockSpec(memory_space=pl.ANY),
        out_shape=jax.ShapeDtypeStruct((n, H, W), x.dtype),
        scratch_shapes=[
            pltpu.VMEM((chunk, H, W), x.dtype),
            pltpu.VMEM((chunk, H, W), x.dtype),
            pltpu.VMEM((chunk, H, W), x.dtype),
            pltpu.VMEM((chunk, H, W), x.dtype),
            pltpu.SemaphoreType.DMA((2,)),
            pltpu.SemaphoreType.DMA((2,)),
        ],
    )(xf)
    return out.reshape(B, C, H, W)


# speedup vs baseline: 1.2192x; 1.0050x over previous
import jax
import jax.numpy as jnp
from jax.experimental import pallas as pl

_P = 5
_PR = _P // 2


def _overlap_add_body(x_ref, out_ref):
    x = x_ref[...]
    h = x.shape[1]
    row = jax.lax.broadcasted_iota(jnp.int32, (1, h, 1), 1)
    acc = jnp.zeros_like(x)
    nvalid = jnp.zeros((1, h, 1), dtype=x.dtype)
    for off in range(-_PR, _PR + 1):
        m = row + off
        ok = jnp.logical_and(m >= _PR, m <= h - 1 - _PR)
        acc = acc + jnp.where(ok, x, 0.0)
        nvalid = nvalid + ok.astype(x.dtype)
    acc = acc * jnp.array(_P, x.dtype)
    counting = nvalid * jnp.array(_P, x.dtype)
    out_ref[...] = acc / counting


def kernel(x):
    B, C, H, W = x.shape
    xf = x.reshape(B * C, H, W)
    bc_block = 96
    grid = (B * C) // bc_block
    out = pl.pallas_call(
        _overlap_add_body,
        grid=(grid,),
        in_specs=[pl.BlockSpec((bc_block, H, W), lambda i: (i, 0, 0))],
        out_specs=pl.BlockSpec((bc_block, H, W), lambda i: (i, 0, 0)),
        out_shape=jax.ShapeDtypeStruct((B * C, H, W), x.dtype),
    )(xf)
    return out.reshape(B, C, H, W)
